# baseline (speedup 1.0000x reference)
import functools

import jax
import jax.numpy as jnp
from jax import lax
from jax.experimental import pallas as pl
from jax.experimental.pallas import tpu as pltpu
from jax.experimental.pallas import tpu_sc as plsc
from jax._src.pallas import mpmd as _mpmd

_B = 1024
_D = 16
_N = 1000000
_TEMP = 0.07
_MOM = 0.5
_EPS = 1e-12

_R = 4096
_NB = -(-_N // _R)
_LOG2E = 1.4426950408889634
_K2 = _LOG2E / _TEMP
_B2 = -_LOG2E / _TEMP
_NC = 2
_NS = 16
_NW = _NC * _NS
_RPW = _B // _NW
_EPW = _RPW * _D


def _tc_body(ft_ref, gt_ref, bankt_ref, gnt_ref, loss_ref,
             fnt_ref, fntb_ref, acc_ref):
    i = pl.program_id(0)
    nb = pl.num_programs(0)

    @pl.when(i == 0)
    def _init():
        ft = ft_ref[...]
        ss = jnp.sum(ft * ft, axis=0, keepdims=True)
        fnt = ft / jnp.maximum(jnp.sqrt(ss), _EPS)
        fnt_ref[...] = fnt
        fntb_ref[...] = fnt.astype(jnp.bfloat16)
        acc_ref[...] = jnp.zeros_like(acc_ref)

    blkt = bankt_ref[...]
    s = lax.dot_general(blkt.astype(jnp.bfloat16), fntb_ref[...],
                        (((0,), (0,)), ((), ())),
                        preferred_element_type=jnp.float32)
    e = jnp.exp2(s * _K2 + _B2)

    @pl.when(i < nb - 1)
    def _accum():
        acc_ref[...] += jnp.sum(e.reshape(_R // 8, 8, _B), axis=0)

    @pl.when(i == nb - 1)
    def _fin():
        rows = lax.broadcasted_iota(jnp.int32, (_R, _B), 0)
        e2 = jnp.where(rows < _N - (_NB - 1) * _R, e, 0.0)
        am = acc_ref[...] + jnp.sum(e2.reshape(_R // 8, 8, _B), axis=0)
        gt = gt_ref[...]
        gss = jnp.sum(gt * gt, axis=0, keepdims=True)
        gnt = gt / jnp.maximum(jnp.sqrt(gss), _EPS)
        gnt_ref[...] = gnt
        pos = jnp.sum(fnt_ref[...] * gnt, axis=0)
        total = jnp.sum(am, axis=0) \
            + jnp.exp((pos - 1.0) * (1.0 / _TEMP))
        lvec = (1.0 / _TEMP) + jnp.log(total) - pos * (1.0 / _TEMP)
        loss_ref[...] = jnp.broadcast_to(jnp.mean(lvec), (1, 1))


def _rsqrt_newton(x):
    xi = lax.bitcast_convert_type(x, jnp.int32)
    yi = jnp.int32(0x5F3759DF) - lax.shift_right_logical(xi, 1)
    y = lax.bitcast_convert_type(yi, jnp.float32)
    for _ in range(3):
        y = y * (1.5 - 0.5 * x * y * y)
    return y


def _sc_body(gnt_hbm, idx_hbm, bank_in, bank_out, idx_v, idx4_v, gnt_v,
             old_v, new_v, sem):
    wid = lax.axis_index("s") * _NC + lax.axis_index("c")
    base = wid * _RPW
    pltpu.sync_copy(idx_hbm.at[pl.ds(base, _RPW)], idx_v)
    pltpu.sync_copy(gnt_hbm.at[:, pl.ds(base, _RPW)], gnt_v)
    for k in range(_D):
        for c in range(_RPW // 16):
            q = k * _RPW + c * 16
            chunk = idx_v[pl.ds(c * 16, 16)] * 16 + jnp.int32(k)
            idx4_v[q // 128, pl.ds(q % 128, 16)] = chunk
    for p in range(_EPW // 128):
        pltpu.async_copy(bank_in.at[idx4_v.at[p]],
                         old_v.at[pl.ds(p * 128, 128)], sem).wait()
    for c in range(_RPW // 16):
        vs = []
        acc = jnp.zeros((16,), jnp.float32)
        for k in range(_D):
            u = (old_v[pl.ds(k * _RPW + c * 16, 16)] * _MOM
                 + gnt_v[k, pl.ds(c * 16, 16)] * (1.0 - _MOM))
            vs.append(u)
            acc += u * u
        rs = _rsqrt_newton(acc)
        for k in range(_D):
            new_v[pl.ds(k * _RPW + c * 16, 16)] = vs[k] * rs
    for p in range(_EPW // 128):
        pltpu.async_copy(new_v.at[pl.ds(p * 128, 128)],
                         bank_out.at[idx4_v.at[p]], sem).wait()


@functools.cache
def _sc_update():
    mesh = plsc.VectorSubcoreMesh(core_axis_name="c", subcore_axis_name="s",
                                  num_cores=_NC, num_subcores=_NS)
    return _mpmd._mpmd_map(
        [(mesh, _sc_body)],
        [jax.ShapeDtypeStruct((_N * _D,), jnp.float32)],
        input_output_aliases={2: 0},
        scratch_types=[
            pltpu.VMEM((_RPW,), jnp.int32),
            pltpu.VMEM((_EPW // 128, 128), jnp.int32),
            pltpu.VMEM((_D, _RPW), jnp.float32),
            pltpu.VMEM((_EPW,), jnp.float32),
            pltpu.VMEM((_EPW,), jnp.float32),
            pltpu.SemaphoreType.DMA,
        ],
        compiler_params=pltpu.CompilerParams(needs_layout_passes=False,
                                             use_tc_tiling_on_sc=False),
    )


_tc_pass = pl.pallas_call(
    _tc_body,
    grid=(_NB,),
    in_specs=[
        pl.BlockSpec((_D, _B), lambda i: (0, 0)),
        pl.BlockSpec((_D, _B), lambda i: (0, 0)),
        pl.BlockSpec((_D, _R), lambda i: (0, i)),
    ],
    out_specs=[
        pl.BlockSpec((_D, _B), lambda i: (0, 0)),
        pl.BlockSpec((1, 1), lambda i: (0, 0)),
    ],
    out_shape=[
        jax.ShapeDtypeStruct((_D, _B), jnp.float32),
        jax.ShapeDtypeStruct((1, 1), jnp.float32),
    ],
    scratch_shapes=[
        pltpu.VMEM((_D, _B), jnp.float32),
        pltpu.VMEM((_D, _B), jnp.bfloat16),
        pltpu.VMEM((8, _B), jnp.float32),
    ],
)


def kernel(f, g, memory_bank, update_idx):
    gnt, loss11 = _tc_pass(f.T, g.T, memory_bank.T)
    bank_flat = jnp.copy(memory_bank).reshape(_N * _D)
    (new_flat,) = _sc_update()(gnt, update_idx, bank_flat)
    return loss11[0, 0], new_flat.reshape(_N, _D)


# speedup vs baseline: 1.0167x; 1.0167x over previous
import functools

import jax
import jax.numpy as jnp
from jax import lax
from jax.experimental import pallas as pl
from jax.experimental.pallas import tpu as pltpu
from jax.experimental.pallas import tpu_sc as plsc
from jax._src.pallas import mpmd as _mpmd

_B = 1024
_D = 16
_N = 1000000
_TEMP = 0.07
_MOM = 0.5
_EPS = 1e-12

_R = 4096
_NB = -(-_N // _R)
_LOG2E = 1.4426950408889634
_K2 = _LOG2E / _TEMP
_B2 = -_LOG2E / _TEMP
_NC = 2
_NS = 16
_NW = _NC * _NS
_RPW = _B // _NW
_EPW = _RPW * _D


def _tc_body(ft_ref, gt_ref, bankt_ref, gnt_ref, loss_ref,
             fnt_ref, fntb_ref, acc_ref):
    i = pl.program_id(0)
    nb = pl.num_programs(0)

    @pl.when(i == 0)
    def _init():
        ft = ft_ref[...]
        ss = jnp.sum(ft * ft, axis=0, keepdims=True)
        fnt = ft / jnp.maximum(jnp.sqrt(ss), _EPS)
        fnt_ref[...] = fnt
        fntb_ref[...] = fnt.astype(jnp.bfloat16)
        acc_ref[...] = jnp.zeros_like(acc_ref)

    blkt = bankt_ref[...]
    s = lax.dot_general(blkt.astype(jnp.bfloat16), fntb_ref[...],
                        (((0,), (0,)), ((), ())),
                        preferred_element_type=jnp.float32)
    e = jnp.exp2(s * _K2 + _B2)

    @pl.when(i < nb - 1)
    def _accum():
        acc_ref[...] += jnp.sum(e.reshape(_R // 8, 8, _B), axis=0)

    @pl.when(i == nb - 1)
    def _fin():
        rows = lax.broadcasted_iota(jnp.int32, (_R, _B), 0)
        e2 = jnp.where(rows < _N - (_NB - 1) * _R, e, 0.0)
        am = acc_ref[...] + jnp.sum(e2.reshape(_R // 8, 8, _B), axis=0)
        gt = gt_ref[...]
        gss = jnp.sum(gt * gt, axis=0, keepdims=True)
        gnt = gt / jnp.maximum(jnp.sqrt(gss), _EPS)
        gnt_ref[...] = gnt
        pos = jnp.sum(fnt_ref[...] * gnt, axis=0)
        total = jnp.sum(am, axis=0) \
            + jnp.exp((pos - 1.0) * (1.0 / _TEMP))
        lvec = (1.0 / _TEMP) + jnp.log(total) - pos * (1.0 / _TEMP)
        loss_ref[...] = jnp.broadcast_to(jnp.mean(lvec), (1, 1))


def _rsqrt_newton(x):
    xi = lax.bitcast_convert_type(x, jnp.int32)
    yi = jnp.int32(0x5F3759DF) - lax.shift_right_logical(xi, 1)
    y = lax.bitcast_convert_type(yi, jnp.float32)
    for _ in range(3):
        y = y * (1.5 - 0.5 * x * y * y)
    return y


def _sc_body(gnt_hbm, idx_hbm, bank_in, bank_out, idx_v, gnt_v, old_v, new_v,
             sem):
    wid = lax.axis_index("s") * _NC + lax.axis_index("c")
    base = wid * _RPW
    pltpu.sync_copy(idx_hbm.at[pl.ds(base, _RPW)], idx_v)
    pltpu.sync_copy(gnt_hbm.at[:, pl.ds(base, _RPW)], gnt_v)
    pltpu.async_copy(bank_in.at[idx_v], old_v, sem).wait()
    for c in range(_RPW // 16):
        rows = lax.iota(jnp.int32, 16) + jnp.int32(c * 16)
        vs = []
        acc = jnp.zeros((16,), jnp.float32)
        for k in range(_D):
            dcol = jnp.full((16,), k, jnp.int32)
            u = (plsc.load_gather(old_v, [rows, dcol]) * _MOM
                 + gnt_v[k, pl.ds(c * 16, 16)] * (1.0 - _MOM))
            vs.append(u)
            acc += u * u
        rs = _rsqrt_newton(acc)
        for k in range(_D):
            dcol = jnp.full((16,), k, jnp.int32)
            plsc.store_scatter(new_v, [rows, dcol], vs[k] * rs)
    pltpu.async_copy(new_v, bank_out.at[idx_v], sem).wait()


@functools.cache
def _sc_update():
    mesh = plsc.VectorSubcoreMesh(core_axis_name="c", subcore_axis_name="s",
                                  num_cores=_NC, num_subcores=_NS)
    return _mpmd._mpmd_map(
        [(mesh, _sc_body)],
        [jax.ShapeDtypeStruct((_N, _D), jnp.float32)],
        input_output_aliases={2: 0},
        scratch_types=[
            pltpu.VMEM((_RPW,), jnp.int32),
            pltpu.VMEM((_D, _RPW), jnp.float32),
            pltpu.VMEM((_RPW, _D), jnp.float32),
            pltpu.VMEM((_RPW, _D), jnp.float32),
            pltpu.SemaphoreType.DMA,
        ],
        compiler_params=pltpu.CompilerParams(needs_layout_passes=False,
                                             use_tc_tiling_on_sc=False),
    )


_tc_pass = pl.pallas_call(
    _tc_body,
    grid=(_NB,),
    in_specs=[
        pl.BlockSpec((_D, _B), lambda i: (0, 0)),
        pl.BlockSpec((_D, _B), lambda i: (0, 0)),
        pl.BlockSpec((_D, _R), lambda i: (0, i)),
    ],
    out_specs=[
        pl.BlockSpec((_D, _B), lambda i: (0, 0)),
        pl.BlockSpec((1, 1), lambda i: (0, 0)),
    ],
    out_shape=[
        jax.ShapeDtypeStruct((_D, _B), jnp.float32),
        jax.ShapeDtypeStruct((1, 1), jnp.float32),
    ],
    scratch_shapes=[
        pltpu.VMEM((_D, _B), jnp.float32),
        pltpu.VMEM((_D, _B), jnp.bfloat16),
        pltpu.VMEM((8, _B), jnp.float32),
    ],
)


def kernel(f, g, memory_bank, update_idx):
    gnt, loss11 = _tc_pass(f.T, g.T, memory_bank.T)
    (new_bank,) = _sc_update()(gnt, update_idx, jnp.copy(memory_bank))
    return loss11[0, 0], new_bank
